# baseline (speedup 1.0000x reference)
import jax
import jax.numpy as jnp
from jax.experimental import pallas as pl
from jax.experimental.pallas import tpu as pltpu

_LEVEL = 2048
_ROWS = 256
_COLS = 8192
_GRID = 2048 // _ROWS


def _masked_indices():
    num_masked = 2
    offsets = jnp.arange(num_masked, dtype=jnp.int32) * _LEVEL
    rkey = jax.random.key(42)
    logits = jnp.ones((_LEVEL,), dtype=jnp.float32)
    slice_ids = jax.random.categorical(rkey, logits, shape=(1, num_masked))
    return (slice_ids.astype(jnp.int32) + offsets[None, :])[0]


def _body(idx_ref, x_ref, o_ref):
    i0 = idx_ref[0]
    c0 = idx_ref[1] * 2
    r0 = pl.program_id(0) * _ROWS
    x = x_ref[...]
    rows = jax.lax.broadcasted_iota(jnp.int32, x.shape, 0) + r0
    cols = jax.lax.broadcasted_iota(jnp.int32, x.shape, 1)
    m = (rows == i0) & ((cols == c0) | (cols == c0 + 1))
    o_ref[...] = jnp.where(m, jnp.float32(0.0), x * jnp.float32(_LEVEL))


def kernel(inputs):
    idx = _masked_indices()
    x2d = inputs.reshape(2048, _COLS)
    out = pl.pallas_call(
        _body,
        grid=(_GRID,),
        in_specs=[
            pl.BlockSpec(memory_space=pltpu.SMEM),
            pl.BlockSpec((_ROWS, _COLS), lambda i: (i, 0)),
        ],
        out_specs=pl.BlockSpec((_ROWS, _COLS), lambda i: (i, 0)),
        out_shape=jax.ShapeDtypeStruct((2048, _COLS), jnp.float32),
    )(idx, x2d)
    return out.reshape(inputs.shape)


# speedup vs baseline: 9.3920x; 9.3920x over previous
import jax
import jax.numpy as jnp
from jax.experimental import pallas as pl
from jax.experimental.pallas import tpu as pltpu

_LEVEL = 2048
_BR = 8192
_NROW = 2048 * 64
_GRID = _NROW // _BR


def _masked_indices():
    num_masked = 2
    offsets = jnp.arange(num_masked, dtype=jnp.int32) * _LEVEL
    rkey = jax.random.key(42)
    logits = jnp.ones((_LEVEL,), dtype=jnp.float32)
    slice_ids = jax.random.categorical(rkey, logits, shape=(1, num_masked))
    return (slice_ids.astype(jnp.int32) + offsets[None, :])[0]


def _body(idx_ref, x_ref, o_ref):
    i0 = idx_ref[0]
    i1 = idx_ref[1]
    g0 = i0 * 64 + (i1 >> 7) * 2
    lane = i1 & 127
    r0 = pl.program_id(0) * _BR
    x = x_ref[...]
    rows = jax.lax.broadcasted_iota(jnp.int32, x.shape, 0) + r0
    lanes = jax.lax.broadcasted_iota(jnp.int32, x.shape, 1)
    m = ((rows == g0) | (rows == g0 + 1)) & (lanes == lane)
    o_ref[...] = jnp.where(m, jnp.float32(0.0), x * jnp.float32(_LEVEL))


def kernel(inputs):
    idx = _masked_indices()
    z = inputs.reshape(2048, 32, 128, 2).transpose(0, 1, 3, 2)
    z = z.reshape(_NROW, 128)
    out = pl.pallas_call(
        _body,
        grid=(_GRID,),
        in_specs=[
            pl.BlockSpec(memory_space=pltpu.SMEM),
            pl.BlockSpec((_BR, 128), lambda i: (i, 0)),
        ],
        out_specs=pl.BlockSpec((_BR, 128), lambda i: (i, 0)),
        out_shape=jax.ShapeDtypeStruct((_NROW, 128), jnp.float32),
    )(idx, z)
    out = out.reshape(2048, 32, 2, 128).transpose(0, 1, 3, 2)
    return out.reshape(2048, 4096, 2)


# speedup vs baseline: 9.5533x; 1.0172x over previous
import jax
import jax.numpy as jnp
from jax.experimental import pallas as pl
from jax.experimental.pallas import tpu as pltpu

_LEVEL = 2048
_BR = 16384
_NROW = 2048 * 64
_GRID = _NROW // _BR


def _masked_indices():
    num_masked = 2
    offsets = jnp.arange(num_masked, dtype=jnp.int32) * _LEVEL
    rkey = jax.random.key(42)
    logits = jnp.ones((_LEVEL,), dtype=jnp.float32)
    slice_ids = jax.random.categorical(rkey, logits, shape=(1, num_masked))
    return (slice_ids.astype(jnp.int32) + offsets[None, :])[0]


def _body(idx_ref, x_ref, o_ref):
    i0 = idx_ref[0]
    i1 = idx_ref[1]
    g0 = i0 * 64 + (i1 >> 7) * 2
    lane = i1 & 127
    r0 = pl.program_id(0) * _BR
    x = x_ref[...]
    rows = jax.lax.broadcasted_iota(jnp.int32, x.shape, 0) + r0
    lanes = jax.lax.broadcasted_iota(jnp.int32, x.shape, 1)
    m = ((rows == g0) | (rows == g0 + 1)) & (lanes == lane)
    o_ref[...] = jnp.where(m, jnp.float32(0.0), x * jnp.float32(_LEVEL))


def kernel(inputs):
    idx = _masked_indices()
    z = inputs.reshape(2048, 32, 128, 2).transpose(0, 1, 3, 2)
    z = z.reshape(_NROW, 128)
    out = pl.pallas_call(
        _body,
        grid=(_GRID,),
        in_specs=[
            pl.BlockSpec(memory_space=pltpu.SMEM),
            pl.BlockSpec((_BR, 128), lambda i: (i, 0)),
        ],
        out_specs=pl.BlockSpec((_BR, 128), lambda i: (i, 0)),
        out_shape=jax.ShapeDtypeStruct((_NROW, 128), jnp.float32),
    )(idx, z)
    out = out.reshape(2048, 32, 2, 128).transpose(0, 1, 3, 2)
    return out.reshape(2048, 4096, 2)
